# baseline (speedup 1.0000x reference)
import functools

import jax
import jax.numpy as jnp
from jax import lax
from jax.experimental import pallas as pl
from jax.experimental.pallas import tpu as pltpu
from jax.experimental.pallas import tpu_sc as plsc

NN = 10000
EE = 320000
DD = 128
NW = 32
EPT = EE // NW
KC = 80
CH = EPT // KC
NP = 10240
RPT = NP // 16

_mesh = plsc.VectorSubcoreMesh(core_axis_name="c", subcore_axis_name="s")


def _sc_degree(dst_r, ones128, z128):
    @functools.partial(
        pl.kernel, mesh=_mesh,
        out_type=jax.ShapeDtypeStruct((2, NP, DD), jnp.float32),
        scratch_types=[
            pltpu.VMEM((CH, KC), jnp.int32),
            pltpu.VMEM((KC, DD), jnp.float32),
            pltpu.VMEM_SHARED((NP, DD), jnp.float32),
            pltpu.SemaphoreType.DMA,
        ],
    )
    def k(dst_hbm, ones_hbm, z_hbm, out_hbm, didx_v, ones_v, acc_sh, sem):
        cid = lax.axis_index("c")
        sid = lax.axis_index("s")
        wid = sid * 2 + cid
        pltpu.sync_copy(z_hbm, acc_sh.at[pl.ds(sid * RPT, RPT)])
        pltpu.sync_copy(dst_hbm.at[wid], didx_v)
        pltpu.sync_copy(ones_hbm, ones_v)
        plsc.subcore_barrier()

        def body(j, c):
            pltpu.sync_copy(ones_v, acc_sh.at[didx_v.at[j]], add=True)
            return c

        lax.fori_loop(0, CH, body, 0)
        plsc.subcore_barrier()
        pltpu.sync_copy(acc_sh.at[pl.ds(sid * RPT, RPT)],
                        out_hbm.at[cid, pl.ds(sid * RPT, RPT)])

    return k(dst_r, ones128, z128)


def _sc_aggregate(g, src_r, dst_r, z128):
    @functools.partial(
        pl.kernel, mesh=_mesh,
        out_type=jax.ShapeDtypeStruct((2, NP, DD), jnp.float32),
        scratch_types=[
            pltpu.VMEM((CH, KC), jnp.int32),
            pltpu.VMEM((CH, KC), jnp.int32),
            pltpu.VMEM((KC, DD), jnp.float32),
            pltpu.VMEM_SHARED((NP, DD), jnp.float32),
            pltpu.SemaphoreType.DMA,
        ],
    )
    def k(g_hbm, src_hbm, dst_hbm, z_hbm, out_hbm,
          sidx_v, didx_v, rows_v, acc_sh, sem):
        cid = lax.axis_index("c")
        sid = lax.axis_index("s")
        wid = sid * 2 + cid
        pltpu.sync_copy(z_hbm, acc_sh.at[pl.ds(sid * RPT, RPT)])
        pltpu.sync_copy(src_hbm.at[wid], sidx_v)
        pltpu.sync_copy(dst_hbm.at[wid], didx_v)
        plsc.subcore_barrier()

        def body(j, c):
            pltpu.sync_copy(g_hbm.at[sidx_v.at[j]], rows_v)
            pltpu.sync_copy(rows_v, acc_sh.at[didx_v.at[j]], add=True)
            return c

        lax.fori_loop(0, CH, body, 0)
        plsc.subcore_barrier()
        pltpu.sync_copy(acc_sh.at[pl.ds(sid * RPT, RPT)],
                        out_hbm.at[cid, pl.ds(sid * RPT, RPT)])

    return k(g, src_r, dst_r, z128)


def _sc_pair_gather(f, src_r, dst_r):
    @functools.partial(
        pl.kernel, mesh=_mesh,
        out_type=(jax.ShapeDtypeStruct((EE, DD), jnp.float32),
                  jax.ShapeDtypeStruct((EE, DD), jnp.float32)),
        scratch_types=[
            pltpu.VMEM((CH, KC), jnp.int32),
            pltpu.VMEM((CH, KC), jnp.int32),
            pltpu.VMEM((KC, DD), jnp.float32),
            pltpu.VMEM((KC, DD), jnp.float32),
            pltpu.SemaphoreType.DMA,
        ],
    )
    def k(f_hbm, src_hbm, dst_hbm, os_hbm, od_hbm,
          sidx_v, didx_v, rs_v, rd_v, sem):
        cid = lax.axis_index("c")
        sid = lax.axis_index("s")
        wid = sid * 2 + cid
        base0 = wid * EPT
        pltpu.sync_copy(src_hbm.at[wid], sidx_v)
        pltpu.sync_copy(dst_hbm.at[wid], didx_v)

        def body(j, c):
            base = pl.multiple_of(base0 + j * KC, 8)
            pltpu.sync_copy(f_hbm.at[sidx_v.at[j]], rs_v)
            pltpu.sync_copy(rs_v, os_hbm.at[pl.ds(base, KC)])
            pltpu.sync_copy(f_hbm.at[didx_v.at[j]], rd_v)
            pltpu.sync_copy(rd_v, od_hbm.at[pl.ds(base, KC)])
            return c

        lax.fori_loop(0, CH, body, 0)

    return k(f, src_r, dst_r)


def _dinv_col(dp_ref):
    deg = dp_ref[0, :, 0:1] + dp_ref[1, :, 0:1] + 1.0
    return lax.rsqrt(deg)


def _tc_g1(x, W1, deg_p):
    R = 1000

    def body(x_ref, w_ref, dp_ref, o_ref):
        col = _dinv_col(dp_ref)
        h = jnp.dot(x_ref[...], w_ref[...], preferred_element_type=jnp.float32)
        o_ref[...] = h * col

    return pl.pallas_call(
        body,
        grid=(NN // R,),
        in_specs=[
            pl.BlockSpec((R, DD), lambda i: (i, 0)),
            pl.BlockSpec((DD, DD), lambda i: (0, 0)),
            pl.BlockSpec((2, R, DD), lambda i: (0, i, 0)),
        ],
        out_specs=pl.BlockSpec((R, DD), lambda i: (i, 0)),
        out_shape=jax.ShapeDtypeStruct((NN, DD), jnp.float32),
    )(x, W1, deg_p)


def _tc_mid(S_p, g1, deg_p, W2, b1):
    R = 1000

    def body(sp_ref, g_ref, dp_ref, w_ref, b_ref, o_ref):
        col = _dinv_col(dp_ref)
        x1 = (sp_ref[0] + sp_ref[1] + g_ref[...]) * col + b_ref[...]
        x1 = jnp.maximum(x1, 0.0)
        h = jnp.dot(x1, w_ref[...], preferred_element_type=jnp.float32)
        o_ref[...] = h * col

    return pl.pallas_call(
        body,
        grid=(NN // R,),
        in_specs=[
            pl.BlockSpec((2, R, DD), lambda i: (0, i, 0)),
            pl.BlockSpec((R, DD), lambda i: (i, 0)),
            pl.BlockSpec((2, R, DD), lambda i: (0, i, 0)),
            pl.BlockSpec((DD, DD), lambda i: (0, 0)),
            pl.BlockSpec((1, DD), lambda i: (0, 0)),
        ],
        out_specs=pl.BlockSpec((R, DD), lambda i: (i, 0)),
        out_shape=jax.ShapeDtypeStruct((NN, DD), jnp.float32),
    )(S_p, g1, deg_p, W2, b1)


def _tc_final(S_p, g2, deg_p, b2, Wc, bc):
    R = 1000
    ncls = Wc.shape[1]

    def body(sp_ref, g_ref, dp_ref, b_ref, wc_ref, bc_ref, f_ref, lg_ref):
        col = _dinv_col(dp_ref)
        f = (sp_ref[0] + sp_ref[1] + g_ref[...]) * col + b_ref[...]
        f_ref[...] = f
        lg_ref[...] = jnp.dot(f, wc_ref[...],
                              preferred_element_type=jnp.float32) + bc_ref[...]

    return pl.pallas_call(
        body,
        grid=(NN // R,),
        in_specs=[
            pl.BlockSpec((2, R, DD), lambda i: (0, i, 0)),
            pl.BlockSpec((R, DD), lambda i: (i, 0)),
            pl.BlockSpec((2, R, DD), lambda i: (0, i, 0)),
            pl.BlockSpec((1, DD), lambda i: (0, 0)),
            pl.BlockSpec((DD, ncls), lambda i: (0, 0)),
            pl.BlockSpec((1, ncls), lambda i: (0, 0)),
        ],
        out_specs=[
            pl.BlockSpec((R, DD), lambda i: (i, 0)),
            pl.BlockSpec((R, ncls), lambda i: (i, 0)),
        ],
        out_shape=[
            jax.ShapeDtypeStruct((NN, DD), jnp.float32),
            jax.ShapeDtypeStruct((NN, ncls), jnp.float32),
        ],
    )(S_p, g2, deg_p, b2, Wc, bc)


def _tc_edge_mlp(fs, fd, Wa, Wb, bp1, Wp2, bp2):
    BK = 1000
    DH = Wa.shape[1]

    def body(fs_ref, fd_ref, a_ref, b_ref, b1_ref, w2_ref, b2_ref, o_ref):
        h = jnp.dot(fs_ref[...], a_ref[...], preferred_element_type=jnp.float32)
        h = h + jnp.dot(fd_ref[...], b_ref[...],
                        preferred_element_type=jnp.float32)
        h = jnp.maximum(h + b1_ref[...], 0.0)
        o_ref[...] = jnp.dot(h, w2_ref[...],
                             preferred_element_type=jnp.float32) + b2_ref[...]

    return pl.pallas_call(
        body,
        grid=(EE // BK,),
        in_specs=[
            pl.BlockSpec((BK, DD), lambda i: (i, 0)),
            pl.BlockSpec((BK, DD), lambda i: (i, 0)),
            pl.BlockSpec((DD, DH), lambda i: (0, 0)),
            pl.BlockSpec((DD, DH), lambda i: (0, 0)),
            pl.BlockSpec((1, DH), lambda i: (0, 0)),
            pl.BlockSpec((DH, DD), lambda i: (0, 0)),
            pl.BlockSpec((1, DD), lambda i: (0, 0)),
        ],
        out_specs=pl.BlockSpec((BK, DD), lambda i: (i, 0)),
        out_shape=jax.ShapeDtypeStruct((EE, DD), jnp.float32),
    )(fs, fd, Wa, Wb, bp1, Wp2, bp2)


def kernel(x, edge_index, W1, b1, W2, b2, Wp1, bp1, Wp2, bp2, Wc, bc):
    src_r = edge_index[0].reshape(NW, CH, KC)
    dst_r = edge_index[1].reshape(NW, CH, KC)
    ones128 = jnp.ones((KC, DD), jnp.float32)
    z128 = jnp.zeros((RPT, DD), jnp.float32)

    deg_p = _sc_degree(dst_r, ones128, z128)
    g1 = _tc_g1(x, W1, deg_p)
    S1 = _sc_aggregate(g1, src_r, dst_r, z128)
    g2 = _tc_mid(S1, g1, deg_p, W2, b1.reshape(1, -1))
    S2 = _sc_aggregate(g2, src_r, dst_r, z128)
    f, logits = _tc_final(S2, g2, deg_p, b2.reshape(1, -1), Wc, bc.reshape(1, -1))
    fs, fd = _sc_pair_gather(f, src_r, dst_r)
    edge_feats = _tc_edge_mlp(fs, fd, Wp1[:DD], Wp1[DD:], bp1.reshape(1, -1),
                              Wp2, bp2.reshape(1, -1))
    return (f, edge_feats, logits, edge_index)


# speedup vs baseline: 1.0733x; 1.0733x over previous
import functools

import jax
import jax.numpy as jnp
from jax import lax
from jax.experimental import pallas as pl
from jax.experimental.pallas import tpu as pltpu
from jax.experimental.pallas import tpu_sc as plsc

NN = 10000
EE = 320000
DD = 128
NW = 32
EPT = EE // NW
KC = 80
CH = EPT // KC
NP = 10240
RPT = NP // 16

_mesh = plsc.VectorSubcoreMesh(core_axis_name="c", subcore_axis_name="s")


def _sc_degree(dst_r, ones128, z128):
    @functools.partial(
        pl.kernel, mesh=_mesh,
        out_type=jax.ShapeDtypeStruct((2, NP, DD), jnp.float32),
        scratch_types=[
            pltpu.VMEM((CH, KC), jnp.int32),
            pltpu.VMEM((KC, DD), jnp.float32),
            pltpu.VMEM_SHARED((NP, DD), jnp.float32),
            pltpu.SemaphoreType.DMA,
        ],
    )
    def k(dst_hbm, ones_hbm, z_hbm, out_hbm, didx_v, ones_v, acc_sh, sem):
        cid = lax.axis_index("c")
        sid = lax.axis_index("s")
        wid = sid * 2 + cid
        pltpu.sync_copy(z_hbm, acc_sh.at[pl.ds(sid * RPT, RPT)])
        pltpu.sync_copy(dst_hbm.at[wid], didx_v)
        pltpu.sync_copy(ones_hbm, ones_v)
        plsc.subcore_barrier()

        def body(j, c):
            pltpu.sync_copy(ones_v, acc_sh.at[didx_v.at[j]], add=True)
            return c

        lax.fori_loop(0, CH, body, 0)
        plsc.subcore_barrier()
        pltpu.sync_copy(acc_sh.at[pl.ds(sid * RPT, RPT)],
                        out_hbm.at[cid, pl.ds(sid * RPT, RPT)])

    return k(dst_r, ones128, z128)


def _sc_aggregate(g, src_r, dst_r, z128):
    @functools.partial(
        pl.kernel, mesh=_mesh,
        out_type=jax.ShapeDtypeStruct((2, NP, DD), jnp.float32),
        scratch_types=[
            pltpu.VMEM((CH, KC), jnp.int32),
            pltpu.VMEM((CH, KC), jnp.int32),
            pltpu.VMEM((KC, DD), jnp.float32),
            pltpu.VMEM_SHARED((NP, DD), jnp.float32),
            pltpu.SemaphoreType.DMA,
        ],
    )
    def k(g_hbm, src_hbm, dst_hbm, z_hbm, out_hbm,
          sidx_v, didx_v, rows_v, acc_sh, sem):
        cid = lax.axis_index("c")
        sid = lax.axis_index("s")
        wid = sid * 2 + cid
        pltpu.sync_copy(z_hbm, acc_sh.at[pl.ds(sid * RPT, RPT)])
        pltpu.sync_copy(src_hbm.at[wid], sidx_v)
        pltpu.sync_copy(dst_hbm.at[wid], didx_v)
        plsc.subcore_barrier()

        def body(j, c):
            pltpu.sync_copy(g_hbm.at[sidx_v.at[j]], rows_v)
            pltpu.sync_copy(rows_v, acc_sh.at[didx_v.at[j]], add=True)
            return c

        lax.fori_loop(0, CH, body, 0)
        plsc.subcore_barrier()
        pltpu.sync_copy(acc_sh.at[pl.ds(sid * RPT, RPT)],
                        out_hbm.at[cid, pl.ds(sid * RPT, RPT)])

    return k(g, src_r, dst_r, z128)


def _sc_pair_gather(f, src_r, dst_r):
    @functools.partial(
        pl.kernel, mesh=_mesh,
        out_type=(jax.ShapeDtypeStruct((EE, DD), jnp.float32),
                  jax.ShapeDtypeStruct((EE, DD), jnp.float32)),
        scratch_types=[
            pltpu.VMEM((CH, KC), jnp.int32),
            pltpu.VMEM((CH, KC), jnp.int32),
            pltpu.VMEM((KC, DD), jnp.float32),
            pltpu.VMEM((KC, DD), jnp.float32),
            pltpu.SemaphoreType.DMA,
        ],
    )
    def k(f_hbm, src_hbm, dst_hbm, os_hbm, od_hbm,
          sidx_v, didx_v, rs_v, rd_v, sem):
        cid = lax.axis_index("c")
        sid = lax.axis_index("s")
        wid = sid * 2 + cid
        base0 = wid * EPT
        pltpu.sync_copy(src_hbm.at[wid], sidx_v)
        pltpu.sync_copy(dst_hbm.at[wid], didx_v)

        def body(j, c):
            base = pl.multiple_of(base0 + j * KC, 8)
            pltpu.sync_copy(f_hbm.at[sidx_v.at[j]], rs_v)
            pltpu.sync_copy(rs_v, os_hbm.at[pl.ds(base, KC)])
            pltpu.sync_copy(f_hbm.at[didx_v.at[j]], rd_v)
            pltpu.sync_copy(rd_v, od_hbm.at[pl.ds(base, KC)])
            return c

        lax.fori_loop(0, CH, body, 0)

    return k(f, src_r, dst_r)


def _dinv_col(dp_ref):
    deg = dp_ref[0, :, 0:1] + dp_ref[1, :, 0:1] + 1.0
    return lax.rsqrt(deg)


def _tc_g1(x, W1, deg_p):
    R = 1000

    def body(x_ref, w_ref, dp_ref, o_ref):
        col = _dinv_col(dp_ref)
        h = jnp.dot(x_ref[...], w_ref[...], preferred_element_type=jnp.float32)
        o_ref[...] = h * col

    return pl.pallas_call(
        body,
        grid=(NN // R,),
        in_specs=[
            pl.BlockSpec((R, DD), lambda i: (i, 0)),
            pl.BlockSpec((DD, DD), lambda i: (0, 0)),
            pl.BlockSpec((2, R, DD), lambda i: (0, i, 0)),
        ],
        out_specs=pl.BlockSpec((R, DD), lambda i: (i, 0)),
        out_shape=jax.ShapeDtypeStruct((NN, DD), jnp.float32),
    )(x, W1, deg_p)


def _tc_mid(S_p, g1, deg_p, W2, b1):
    R = 1000

    def body(sp_ref, g_ref, dp_ref, w_ref, b_ref, o_ref):
        col = _dinv_col(dp_ref)
        x1 = (sp_ref[0] + sp_ref[1] + g_ref[...]) * col + b_ref[...]
        x1 = jnp.maximum(x1, 0.0)
        h = jnp.dot(x1, w_ref[...], preferred_element_type=jnp.float32)
        o_ref[...] = h * col

    return pl.pallas_call(
        body,
        grid=(NN // R,),
        in_specs=[
            pl.BlockSpec((2, R, DD), lambda i: (0, i, 0)),
            pl.BlockSpec((R, DD), lambda i: (i, 0)),
            pl.BlockSpec((2, R, DD), lambda i: (0, i, 0)),
            pl.BlockSpec((DD, DD), lambda i: (0, 0)),
            pl.BlockSpec((1, DD), lambda i: (0, 0)),
        ],
        out_specs=pl.BlockSpec((R, DD), lambda i: (i, 0)),
        out_shape=jax.ShapeDtypeStruct((NN, DD), jnp.float32),
    )(S_p, g1, deg_p, W2, b1)


def _tc_final(S_p, g2, deg_p, b2, Wc, bc):
    R = 1000
    ncls = Wc.shape[1]

    def body(sp_ref, g_ref, dp_ref, b_ref, wc_ref, bc_ref, f_ref, fb_ref, lg_ref):
        col = _dinv_col(dp_ref)
        f = (sp_ref[0] + sp_ref[1] + g_ref[...]) * col + b_ref[...]
        f_ref[...] = f
        fb_ref[...] = f.astype(jnp.bfloat16)
        lg_ref[...] = jnp.dot(f, wc_ref[...],
                              preferred_element_type=jnp.float32) + bc_ref[...]

    return pl.pallas_call(
        body,
        grid=(NN // R,),
        in_specs=[
            pl.BlockSpec((2, R, DD), lambda i: (0, i, 0)),
            pl.BlockSpec((R, DD), lambda i: (i, 0)),
            pl.BlockSpec((2, R, DD), lambda i: (0, i, 0)),
            pl.BlockSpec((1, DD), lambda i: (0, 0)),
            pl.BlockSpec((DD, ncls), lambda i: (0, 0)),
            pl.BlockSpec((1, ncls), lambda i: (0, 0)),
        ],
        out_specs=[
            pl.BlockSpec((R, DD), lambda i: (i, 0)),
            pl.BlockSpec((R, DD), lambda i: (i, 0)),
            pl.BlockSpec((R, ncls), lambda i: (i, 0)),
        ],
        out_shape=[
            jax.ShapeDtypeStruct((NN, DD), jnp.float32),
            jax.ShapeDtypeStruct((NN, DD), jnp.bfloat16),
            jax.ShapeDtypeStruct((NN, ncls), jnp.float32),
        ],
    )(S_p, g2, deg_p, b2, Wc, bc)


def _tc_edge_mlp(fs, fd, Wa, Wb, bp1, Wp2, bp2):
    BK = 2000
    DH = Wa.shape[1]

    def body(fs_ref, fd_ref, a_ref, b_ref, b1_ref, w2_ref, b2_ref, o_ref):
        h = jnp.dot(fs_ref[...].astype(jnp.bfloat16), a_ref[...],
                    preferred_element_type=jnp.float32)
        h = h + jnp.dot(fd_ref[...].astype(jnp.bfloat16), b_ref[...],
                        preferred_element_type=jnp.float32)
        h = jnp.maximum(h + b1_ref[...], 0.0).astype(jnp.bfloat16)
        o_ref[...] = jnp.dot(h, w2_ref[...],
                             preferred_element_type=jnp.float32) + b2_ref[...]

    return pl.pallas_call(
        body,
        grid=(EE // BK,),
        in_specs=[
            pl.BlockSpec((BK, DD), lambda i: (i, 0)),
            pl.BlockSpec((BK, DD), lambda i: (i, 0)),
            pl.BlockSpec((DD, DH), lambda i: (0, 0)),
            pl.BlockSpec((DD, DH), lambda i: (0, 0)),
            pl.BlockSpec((1, DH), lambda i: (0, 0)),
            pl.BlockSpec((DH, DD), lambda i: (0, 0)),
            pl.BlockSpec((1, DD), lambda i: (0, 0)),
        ],
        out_specs=pl.BlockSpec((BK, DD), lambda i: (i, 0)),
        out_shape=jax.ShapeDtypeStruct((EE, DD), jnp.float32),
    )(fs, fd, Wa, Wb, bp1, Wp2, bp2)


def kernel(x, edge_index, W1, b1, W2, b2, Wp1, bp1, Wp2, bp2, Wc, bc):
    src_r = edge_index[0].reshape(NW, CH, KC)
    dst_r = edge_index[1].reshape(NW, CH, KC)
    ones128 = jnp.ones((KC, DD), jnp.float32)
    z128 = jnp.zeros((RPT, DD), jnp.float32)

    deg_p = _sc_degree(dst_r, ones128, z128)
    g1 = _tc_g1(x, W1, deg_p)
    S1 = _sc_aggregate(g1, src_r, dst_r, z128)
    g2 = _tc_mid(S1, g1, deg_p, W2, b1.reshape(1, -1))
    S2 = _sc_aggregate(g2, src_r, dst_r, z128)
    f, f_bf, logits = _tc_final(S2, g2, deg_p, b2.reshape(1, -1), Wc,
                                bc.reshape(1, -1))
    fs, fd = _sc_pair_gather(f, src_r, dst_r)
    edge_feats = _tc_edge_mlp(fs, fd,
                              Wp1[:DD].astype(jnp.bfloat16),
                              Wp1[DD:].astype(jnp.bfloat16),
                              bp1.reshape(1, -1),
                              Wp2.astype(jnp.bfloat16), bp2.reshape(1, -1))
    return (f, edge_feats, logits, edge_index)


# speedup vs baseline: 1.2194x; 1.1361x over previous
import functools

import jax
import jax.numpy as jnp
from jax import lax
from jax.experimental import pallas as pl
from jax.experimental.pallas import tpu as pltpu
from jax.experimental.pallas import tpu_sc as plsc

NN = 10000
EE = 320000
DD = 128
NW = 32
EPT = EE // NW
KC = 80
CH = EPT // KC
NP = 10240
RPT = NP // 16

_mesh = plsc.VectorSubcoreMesh(core_axis_name="c", subcore_axis_name="s")


def _sc_degree(dst_r, ones128, z128):
    @functools.partial(
        pl.kernel, mesh=_mesh,
        out_type=jax.ShapeDtypeStruct((2, NP, DD), jnp.float32),
        scratch_types=[
            pltpu.VMEM((CH, KC), jnp.int32),
            pltpu.VMEM((KC, DD), jnp.float32),
            pltpu.VMEM_SHARED((NP, DD), jnp.float32),
            pltpu.SemaphoreType.DMA,
        ],
    )
    def k(dst_hbm, ones_hbm, z_hbm, out_hbm, didx_v, ones_v, acc_sh, sem):
        cid = lax.axis_index("c")
        sid = lax.axis_index("s")
        wid = sid * 2 + cid
        pltpu.sync_copy(z_hbm, acc_sh.at[pl.ds(sid * RPT, RPT)])
        pltpu.sync_copy(dst_hbm.at[wid], didx_v)
        pltpu.sync_copy(ones_hbm, ones_v)
        plsc.subcore_barrier()

        def body(j, c):
            pltpu.sync_copy(ones_v, acc_sh.at[didx_v.at[j]], add=True)
            return c

        lax.fori_loop(0, CH, body, 0)
        plsc.subcore_barrier()
        pltpu.sync_copy(acc_sh.at[pl.ds(sid * RPT, RPT)],
                        out_hbm.at[cid, pl.ds(sid * RPT, RPT)])

    return k(dst_r, ones128, z128)


def _sc_aggregate(g, src_r, dst_r, z128):
    @functools.partial(
        pl.kernel, mesh=_mesh,
        out_type=jax.ShapeDtypeStruct((2, NP, DD), jnp.float32),
        scratch_types=[
            pltpu.VMEM((CH, KC), jnp.int32),
            pltpu.VMEM((CH, KC), jnp.int32),
            pltpu.VMEM((KC, DD), jnp.float32),
            pltpu.VMEM_SHARED((NP, DD), jnp.float32),
            pltpu.SemaphoreType.DMA,
        ],
    )
    def k(g_hbm, src_hbm, dst_hbm, z_hbm, out_hbm,
          sidx_v, didx_v, rows_v, acc_sh, sem):
        cid = lax.axis_index("c")
        sid = lax.axis_index("s")
        wid = sid * 2 + cid
        pltpu.sync_copy(z_hbm, acc_sh.at[pl.ds(sid * RPT, RPT)])
        pltpu.sync_copy(src_hbm.at[wid], sidx_v)
        pltpu.sync_copy(dst_hbm.at[wid], didx_v)
        plsc.subcore_barrier()

        def body(j, c):
            pltpu.sync_copy(g_hbm.at[sidx_v.at[j]], rows_v)
            pltpu.sync_copy(rows_v, acc_sh.at[didx_v.at[j]], add=True)
            return c

        lax.fori_loop(0, CH, body, 0)
        plsc.subcore_barrier()
        pltpu.sync_copy(acc_sh.at[pl.ds(sid * RPT, RPT)],
                        out_hbm.at[cid, pl.ds(sid * RPT, RPT)])

    return k(g, src_r, dst_r, z128)


def _sc_pair_gather(f, src_r, dst_r):
    @functools.partial(
        pl.kernel, mesh=_mesh,
        out_type=(jax.ShapeDtypeStruct((EE, DD), jnp.float32),
                  jax.ShapeDtypeStruct((EE, DD), jnp.float32)),
        scratch_types=[
            pltpu.VMEM((CH, KC), jnp.int32),
            pltpu.VMEM((CH, KC), jnp.int32),
            pltpu.VMEM((KC, DD), jnp.float32),
            pltpu.VMEM((KC, DD), jnp.float32),
            pltpu.VMEM((KC, DD), jnp.float32),
            pltpu.VMEM((KC, DD), jnp.float32),
            pltpu.SemaphoreType.DMA,
            pltpu.SemaphoreType.DMA,
        ],
    )
    def k(f_hbm, src_hbm, dst_hbm, os_hbm, od_hbm,
          sidx_v, didx_v, rs0_v, rd0_v, rs1_v, rd1_v, sem0, sem1):
        cid = lax.axis_index("c")
        sid = lax.axis_index("s")
        wid = sid * 2 + cid
        base0 = wid * EPT
        pltpu.sync_copy(src_hbm.at[wid], sidx_v)
        pltpu.sync_copy(dst_hbm.at[wid], didx_v)

        def body(jj, c):
            j = 2 * jj
            cs0 = pltpu.async_copy(f_hbm.at[sidx_v.at[j]], rs0_v, sem0)
            cd0 = pltpu.async_copy(f_hbm.at[didx_v.at[j]], rd0_v, sem0)
            cs1 = pltpu.async_copy(f_hbm.at[sidx_v.at[j + 1]], rs1_v, sem1)
            cd1 = pltpu.async_copy(f_hbm.at[didx_v.at[j + 1]], rd1_v, sem1)
            base = pl.multiple_of(base0 + j * KC, 8)
            base1 = pl.multiple_of(base0 + (j + 1) * KC, 8)
            cs0.wait()
            cd0.wait()
            pltpu.sync_copy(rs0_v, os_hbm.at[pl.ds(base, KC)])
            pltpu.sync_copy(rd0_v, od_hbm.at[pl.ds(base, KC)])
            cs1.wait()
            cd1.wait()
            pltpu.sync_copy(rs1_v, os_hbm.at[pl.ds(base1, KC)])
            pltpu.sync_copy(rd1_v, od_hbm.at[pl.ds(base1, KC)])
            return c

        lax.fori_loop(0, CH // 2, body, 0)
        base = pl.multiple_of(base0 + (CH - 1) * KC, 8)
        cs = pltpu.async_copy(f_hbm.at[sidx_v.at[CH - 1]], rs0_v, sem0)
        cd = pltpu.async_copy(f_hbm.at[didx_v.at[CH - 1]], rd0_v, sem0)
        cs.wait()
        cd.wait()
        pltpu.sync_copy(rs0_v, os_hbm.at[pl.ds(base, KC)])
        pltpu.sync_copy(rd0_v, od_hbm.at[pl.ds(base, KC)])

    return k(f, src_r, dst_r)


def _dinv_col(dp_ref):
    deg = dp_ref[0, :, 0:1] + dp_ref[1, :, 0:1] + 1.0
    return lax.rsqrt(deg)


def _tc_g1(x, W1, deg_p):
    R = 1000

    def body(x_ref, w_ref, dp_ref, o_ref):
        col = _dinv_col(dp_ref)
        h = jnp.dot(x_ref[...], w_ref[...], preferred_element_type=jnp.float32)
        o_ref[...] = h * col

    return pl.pallas_call(
        body,
        grid=(NN // R,),
        in_specs=[
            pl.BlockSpec((R, DD), lambda i: (i, 0)),
            pl.BlockSpec((DD, DD), lambda i: (0, 0)),
            pl.BlockSpec((2, R, DD), lambda i: (0, i, 0)),
        ],
        out_specs=pl.BlockSpec((R, DD), lambda i: (i, 0)),
        out_shape=jax.ShapeDtypeStruct((NN, DD), jnp.float32),
    )(x, W1, deg_p)


def _tc_mid(S_p, g1, deg_p, W2, b1):
    R = 1000

    def body(sp_ref, g_ref, dp_ref, w_ref, b_ref, o_ref):
        col = _dinv_col(dp_ref)
        x1 = (sp_ref[0] + sp_ref[1] + g_ref[...]) * col + b_ref[...]
        x1 = jnp.maximum(x1, 0.0)
        h = jnp.dot(x1, w_ref[...], preferred_element_type=jnp.float32)
        o_ref[...] = h * col

    return pl.pallas_call(
        body,
        grid=(NN // R,),
        in_specs=[
            pl.BlockSpec((2, R, DD), lambda i: (0, i, 0)),
            pl.BlockSpec((R, DD), lambda i: (i, 0)),
            pl.BlockSpec((2, R, DD), lambda i: (0, i, 0)),
            pl.BlockSpec((DD, DD), lambda i: (0, 0)),
            pl.BlockSpec((1, DD), lambda i: (0, 0)),
        ],
        out_specs=pl.BlockSpec((R, DD), lambda i: (i, 0)),
        out_shape=jax.ShapeDtypeStruct((NN, DD), jnp.float32),
    )(S_p, g1, deg_p, W2, b1)


def _tc_final(S_p, g2, deg_p, b2, Wc, bc):
    R = 1000
    ncls = Wc.shape[1]

    def body(sp_ref, g_ref, dp_ref, b_ref, wc_ref, bc_ref, f_ref, fb_ref, lg_ref):
        col = _dinv_col(dp_ref)
        f = (sp_ref[0] + sp_ref[1] + g_ref[...]) * col + b_ref[...]
        f_ref[...] = f
        fb_ref[...] = f.astype(jnp.bfloat16)
        lg_ref[...] = jnp.dot(f, wc_ref[...],
                              preferred_element_type=jnp.float32) + bc_ref[...]

    return pl.pallas_call(
        body,
        grid=(NN // R,),
        in_specs=[
            pl.BlockSpec((2, R, DD), lambda i: (0, i, 0)),
            pl.BlockSpec((R, DD), lambda i: (i, 0)),
            pl.BlockSpec((2, R, DD), lambda i: (0, i, 0)),
            pl.BlockSpec((1, DD), lambda i: (0, 0)),
            pl.BlockSpec((DD, ncls), lambda i: (0, 0)),
            pl.BlockSpec((1, ncls), lambda i: (0, 0)),
        ],
        out_specs=[
            pl.BlockSpec((R, DD), lambda i: (i, 0)),
            pl.BlockSpec((R, DD), lambda i: (i, 0)),
            pl.BlockSpec((R, ncls), lambda i: (i, 0)),
        ],
        out_shape=[
            jax.ShapeDtypeStruct((NN, DD), jnp.float32),
            jax.ShapeDtypeStruct((NN, DD), jnp.bfloat16),
            jax.ShapeDtypeStruct((NN, ncls), jnp.float32),
        ],
    )(S_p, g2, deg_p, b2, Wc, bc)


def _tc_edge_mlp(fs, fd, Wa, Wb, bp1, Wp2, bp2):
    BK = 2000
    DH = Wa.shape[1]

    def body(fs_ref, fd_ref, a_ref, b_ref, b1_ref, w2_ref, b2_ref, o_ref):
        h = jnp.dot(fs_ref[...].astype(jnp.bfloat16), a_ref[...],
                    preferred_element_type=jnp.float32)
        h = h + jnp.dot(fd_ref[...].astype(jnp.bfloat16), b_ref[...],
                        preferred_element_type=jnp.float32)
        h = jnp.maximum(h + b1_ref[...], 0.0).astype(jnp.bfloat16)
        o_ref[...] = jnp.dot(h, w2_ref[...],
                             preferred_element_type=jnp.float32) + b2_ref[...]

    return pl.pallas_call(
        body,
        grid=(EE // BK,),
        in_specs=[
            pl.BlockSpec((BK, DD), lambda i: (i, 0)),
            pl.BlockSpec((BK, DD), lambda i: (i, 0)),
            pl.BlockSpec((DD, DH), lambda i: (0, 0)),
            pl.BlockSpec((DD, DH), lambda i: (0, 0)),
            pl.BlockSpec((1, DH), lambda i: (0, 0)),
            pl.BlockSpec((DH, DD), lambda i: (0, 0)),
            pl.BlockSpec((1, DD), lambda i: (0, 0)),
        ],
        out_specs=pl.BlockSpec((BK, DD), lambda i: (i, 0)),
        out_shape=jax.ShapeDtypeStruct((EE, DD), jnp.float32),
    )(fs, fd, Wa, Wb, bp1, Wp2, bp2)


def kernel(x, edge_index, W1, b1, W2, b2, Wp1, bp1, Wp2, bp2, Wc, bc):
    src_r = edge_index[0].reshape(NW, CH, KC)
    dst_r = edge_index[1].reshape(NW, CH, KC)
    ones128 = jnp.ones((KC, DD), jnp.float32)
    z128 = jnp.zeros((RPT, DD), jnp.float32)

    deg_p = _sc_degree(dst_r, ones128, z128)
    g1 = _tc_g1(x, W1, deg_p)
    S1 = _sc_aggregate(g1, src_r, dst_r, z128)
    g2 = _tc_mid(S1, g1, deg_p, W2, b1.reshape(1, -1))
    S2 = _sc_aggregate(g2, src_r, dst_r, z128)
    f, f_bf, logits = _tc_final(S2, g2, deg_p, b2.reshape(1, -1), Wc,
                                bc.reshape(1, -1))
    fs, fd = _sc_pair_gather(f, src_r, dst_r)
    edge_feats = _tc_edge_mlp(fs, fd,
                              Wp1[:DD].astype(jnp.bfloat16),
                              Wp1[DD:].astype(jnp.bfloat16),
                              bp1.reshape(1, -1),
                              Wp2.astype(jnp.bfloat16), bp2.reshape(1, -1))
    return (f, edge_feats, logits, edge_index)


# speedup vs baseline: 1.2649x; 1.0373x over previous
import functools

import jax
import jax.numpy as jnp
from jax import lax
from jax.experimental import pallas as pl
from jax.experimental.pallas import tpu as pltpu
from jax.experimental.pallas import tpu_sc as plsc

NN = 10000
EE = 320000
DD = 128
NW = 32
EPT = EE // NW
KC = 80
CH = EPT // KC
NP = 10240
RPT = NP // 16

CHA = 63
CHB = CH - CHA
EA = NW * CHA * KC
EB = EE - EA
BKA = 2016
BKB = 1280

_mesh = plsc.VectorSubcoreMesh(core_axis_name="c", subcore_axis_name="s")


def _sc_degree(dst_r, ones128, z128):
    @functools.partial(
        pl.kernel, mesh=_mesh,
        out_type=jax.ShapeDtypeStruct((2, NP, DD), jnp.float32),
        scratch_types=[
            pltpu.VMEM((CH, KC), jnp.int32),
            pltpu.VMEM((KC, DD), jnp.float32),
            pltpu.VMEM_SHARED((NP, DD), jnp.float32),
            pltpu.SemaphoreType.DMA,
        ],
    )
    def k(dst_hbm, ones_hbm, z_hbm, out_hbm, didx_v, ones_v, acc_sh, sem):
        cid = lax.axis_index("c")
        sid = lax.axis_index("s")
        wid = sid * 2 + cid
        pltpu.sync_copy(z_hbm, acc_sh.at[pl.ds(sid * RPT, RPT)])
        pltpu.sync_copy(dst_hbm.at[wid], didx_v)
        pltpu.sync_copy(ones_hbm, ones_v)
        plsc.subcore_barrier()

        def body(j, c):
            pltpu.sync_copy(ones_v, acc_sh.at[didx_v.at[j]], add=True)
            return c

        lax.fori_loop(0, CH, body, 0)
        plsc.subcore_barrier()
        pltpu.sync_copy(acc_sh.at[pl.ds(sid * RPT, RPT)],
                        out_hbm.at[cid, pl.ds(sid * RPT, RPT)])

    return k(dst_r, ones128, z128)


def _sc_aggregate(g, src_r, dst_r, z128):
    @functools.partial(
        pl.kernel, mesh=_mesh,
        out_type=jax.ShapeDtypeStruct((2, NP, DD), jnp.float32),
        scratch_types=[
            pltpu.VMEM((CH, KC), jnp.int32),
            pltpu.VMEM((CH, KC), jnp.int32),
            pltpu.VMEM((KC, DD), jnp.float32),
            pltpu.VMEM_SHARED((NP, DD), jnp.float32),
            pltpu.SemaphoreType.DMA,
        ],
    )
    def k(g_hbm, src_hbm, dst_hbm, z_hbm, out_hbm,
          sidx_v, didx_v, rows_v, acc_sh, sem):
        cid = lax.axis_index("c")
        sid = lax.axis_index("s")
        wid = sid * 2 + cid
        pltpu.sync_copy(z_hbm, acc_sh.at[pl.ds(sid * RPT, RPT)])
        pltpu.sync_copy(src_hbm.at[wid], sidx_v)
        pltpu.sync_copy(dst_hbm.at[wid], didx_v)
        plsc.subcore_barrier()

        def body(j, c):
            pltpu.sync_copy(g_hbm.at[sidx_v.at[j]], rows_v)
            pltpu.sync_copy(rows_v, acc_sh.at[didx_v.at[j]], add=True)
            return c

        lax.fori_loop(0, CH, body, 0)
        plsc.subcore_barrier()
        pltpu.sync_copy(acc_sh.at[pl.ds(sid * RPT, RPT)],
                        out_hbm.at[cid, pl.ds(sid * RPT, RPT)])

    return k(g, src_r, dst_r, z128)


def _sc_pair_gather(f, src_r, dst_r, ch):
    ne = NW * ch * KC

    @functools.partial(
        pl.kernel, mesh=_mesh,
        out_type=(jax.ShapeDtypeStruct((ne, DD), jnp.float32),
                  jax.ShapeDtypeStruct((ne, DD), jnp.float32)),
        scratch_types=[
            pltpu.VMEM((ch, KC), jnp.int32),
            pltpu.VMEM((ch, KC), jnp.int32),
            pltpu.VMEM((KC, DD), jnp.float32),
            pltpu.VMEM((KC, DD), jnp.float32),
            pltpu.VMEM((KC, DD), jnp.float32),
            pltpu.VMEM((KC, DD), jnp.float32),
            pltpu.SemaphoreType.DMA,
            pltpu.SemaphoreType.DMA,
        ],
    )
    def k(f_hbm, src_hbm, dst_hbm, os_hbm, od_hbm,
          sidx_v, didx_v, rs0_v, rd0_v, rs1_v, rd1_v, sem0, sem1):
        cid = lax.axis_index("c")
        sid = lax.axis_index("s")
        wid = sid * 2 + cid
        base0 = wid * KC
        pltpu.sync_copy(src_hbm.at[wid], sidx_v)
        pltpu.sync_copy(dst_hbm.at[wid], didx_v)

        def body(jj, c):
            j = 2 * jj
            cs0 = pltpu.async_copy(f_hbm.at[sidx_v.at[j]], rs0_v, sem0)
            cd0 = pltpu.async_copy(f_hbm.at[didx_v.at[j]], rd0_v, sem0)
            cs1 = pltpu.async_copy(f_hbm.at[sidx_v.at[j + 1]], rs1_v, sem1)
            cd1 = pltpu.async_copy(f_hbm.at[didx_v.at[j + 1]], rd1_v, sem1)
            base = pl.multiple_of(base0 + j * NW * KC, 8)
            base1 = pl.multiple_of(base0 + (j + 1) * NW * KC, 8)
            cs0.wait()
            cd0.wait()
            pltpu.sync_copy(rs0_v, os_hbm.at[pl.ds(base, KC)])
            pltpu.sync_copy(rd0_v, od_hbm.at[pl.ds(base, KC)])
            cs1.wait()
            cd1.wait()
            pltpu.sync_copy(rs1_v, os_hbm.at[pl.ds(base1, KC)])
            pltpu.sync_copy(rd1_v, od_hbm.at[pl.ds(base1, KC)])
            return c

        lax.fori_loop(0, ch // 2, body, 0)
        if ch % 2:
            base = pl.multiple_of(base0 + (ch - 1) * NW * KC, 8)
            cs = pltpu.async_copy(f_hbm.at[sidx_v.at[ch - 1]], rs0_v, sem0)
            cd = pltpu.async_copy(f_hbm.at[didx_v.at[ch - 1]], rd0_v, sem0)
            cs.wait()
            cd.wait()
            pltpu.sync_copy(rs0_v, os_hbm.at[pl.ds(base, KC)])
            pltpu.sync_copy(rd0_v, od_hbm.at[pl.ds(base, KC)])

    return k(f, src_r, dst_r)


def _dinv_col(dp_ref):
    deg = dp_ref[0, :, 0:1] + dp_ref[1, :, 0:1] + 1.0
    return lax.rsqrt(deg)


def _tc_g1(x, W1, deg_p):
    R = 1000

    def body(x_ref, w_ref, dp_ref, o_ref):
        col = _dinv_col(dp_ref)
        h = jnp.dot(x_ref[...], w_ref[...], preferred_element_type=jnp.float32)
        o_ref[...] = h * col

    return pl.pallas_call(
        body,
        grid=(NN // R,),
        in_specs=[
            pl.BlockSpec((R, DD), lambda i: (i, 0)),
            pl.BlockSpec((DD, DD), lambda i: (0, 0)),
            pl.BlockSpec((2, R, DD), lambda i: (0, i, 0)),
        ],
        out_specs=pl.BlockSpec((R, DD), lambda i: (i, 0)),
        out_shape=jax.ShapeDtypeStruct((NN, DD), jnp.float32),
    )(x, W1, deg_p)


def _tc_mid(S_p, g1, deg_p, W2, b1):
    R = 1000

    def body(sp_ref, g_ref, dp_ref, w_ref, b_ref, o_ref):
        col = _dinv_col(dp_ref)
        x1 = (sp_ref[0] + sp_ref[1] + g_ref[...]) * col + b_ref[...]
        x1 = jnp.maximum(x1, 0.0)
        h = jnp.dot(x1, w_ref[...], preferred_element_type=jnp.float32)
        o_ref[...] = h * col

    return pl.pallas_call(
        body,
        grid=(NN // R,),
        in_specs=[
            pl.BlockSpec((2, R, DD), lambda i: (0, i, 0)),
            pl.BlockSpec((R, DD), lambda i: (i, 0)),
            pl.BlockSpec((2, R, DD), lambda i: (0, i, 0)),
            pl.BlockSpec((DD, DD), lambda i: (0, 0)),
            pl.BlockSpec((1, DD), lambda i: (0, 0)),
        ],
        out_specs=pl.BlockSpec((R, DD), lambda i: (i, 0)),
        out_shape=jax.ShapeDtypeStruct((NN, DD), jnp.float32),
    )(S_p, g1, deg_p, W2, b1)


def _tc_final(S_p, g2, deg_p, b2, Wc, bc):
    R = 1000
    ncls = Wc.shape[1]

    def body(sp_ref, g_ref, dp_ref, b_ref, wc_ref, bc_ref, f_ref, lg_ref):
        col = _dinv_col(dp_ref)
        f = (sp_ref[0] + sp_ref[1] + g_ref[...]) * col + b_ref[...]
        f_ref[...] = f
        lg_ref[...] = jnp.dot(f, wc_ref[...],
                              preferred_element_type=jnp.float32) + bc_ref[...]

    return pl.pallas_call(
        body,
        grid=(NN // R,),
        in_specs=[
            pl.BlockSpec((2, R, DD), lambda i: (0, i, 0)),
            pl.BlockSpec((R, DD), lambda i: (i, 0)),
            pl.BlockSpec((2, R, DD), lambda i: (0, i, 0)),
            pl.BlockSpec((1, DD), lambda i: (0, 0)),
            pl.BlockSpec((DD, ncls), lambda i: (0, 0)),
            pl.BlockSpec((1, ncls), lambda i: (0, 0)),
        ],
        out_specs=[
            pl.BlockSpec((R, DD), lambda i: (i, 0)),
            pl.BlockSpec((R, ncls), lambda i: (i, 0)),
        ],
        out_shape=[
            jax.ShapeDtypeStruct((NN, DD), jnp.float32),
            jax.ShapeDtypeStruct((NN, ncls), jnp.float32),
        ],
    )(S_p, g2, deg_p, b2, Wc, bc)


def _edge_mlp_body(fs_ref, fd_ref, a_ref, b_ref, b1_ref, w2_ref, b2_ref, o_ref):
    h = jnp.dot(fs_ref[...].astype(jnp.bfloat16), a_ref[...],
                preferred_element_type=jnp.float32)
    h = h + jnp.dot(fd_ref[...].astype(jnp.bfloat16), b_ref[...],
                    preferred_element_type=jnp.float32)
    h = jnp.maximum(h + b1_ref[...], 0.0).astype(jnp.bfloat16)
    o_ref[...] = jnp.dot(h, w2_ref[...],
                         preferred_element_type=jnp.float32) + b2_ref[...]


def _tc_edge_mlp_a(fs, fd, Wa, Wb, bp1, Wp2, bp2):
    DH = Wa.shape[1]

    return pl.pallas_call(
        _edge_mlp_body,
        grid=(EA // BKA,),
        in_specs=[
            pl.BlockSpec((BKA, DD), lambda i: (i, 0)),
            pl.BlockSpec((BKA, DD), lambda i: (i, 0)),
            pl.BlockSpec((DD, DH), lambda i: (0, 0)),
            pl.BlockSpec((DD, DH), lambda i: (0, 0)),
            pl.BlockSpec((1, DH), lambda i: (0, 0)),
            pl.BlockSpec((DH, DD), lambda i: (0, 0)),
            pl.BlockSpec((1, DD), lambda i: (0, 0)),
        ],
        out_specs=pl.BlockSpec((BKA, DD), lambda i: (i, 0)),
        out_shape=jax.ShapeDtypeStruct((EE, DD), jnp.float32),
    )(fs, fd, Wa, Wb, bp1, Wp2, bp2)


def _tc_edge_mlp_b(prev, fs, fd, Wa, Wb, bp1, Wp2, bp2):
    DH = Wa.shape[1]
    OFF = EA // BKB

    def body(p_ref, fs_ref, fd_ref, a_ref, b_ref, b1_ref, w2_ref, b2_ref,
             o_ref):
        _edge_mlp_body(fs_ref, fd_ref, a_ref, b_ref, b1_ref, w2_ref, b2_ref,
                       o_ref)

    return pl.pallas_call(
        body,
        grid=(EB // BKB,),
        in_specs=[
            pl.BlockSpec(memory_space=pl.ANY),
            pl.BlockSpec((BKB, DD), lambda i: (i, 0)),
            pl.BlockSpec((BKB, DD), lambda i: (i, 0)),
            pl.BlockSpec((DD, DH), lambda i: (0, 0)),
            pl.BlockSpec((DD, DH), lambda i: (0, 0)),
            pl.BlockSpec((1, DH), lambda i: (0, 0)),
            pl.BlockSpec((DH, DD), lambda i: (0, 0)),
            pl.BlockSpec((1, DD), lambda i: (0, 0)),
        ],
        out_specs=pl.BlockSpec((BKB, DD), lambda i: (i + OFF, 0)),
        out_shape=jax.ShapeDtypeStruct((EE, DD), jnp.float32),
        input_output_aliases={0: 0},
    )(prev, fs, fd, Wa, Wb, bp1, Wp2, bp2)


def kernel(x, edge_index, W1, b1, W2, b2, Wp1, bp1, Wp2, bp2, Wc, bc):
    src_r = edge_index[0].reshape(NW, CH, KC)
    dst_r = edge_index[1].reshape(NW, CH, KC)
    src_a = edge_index[0][:EA].reshape(CHA, NW, KC).transpose(1, 0, 2)
    dst_a = edge_index[1][:EA].reshape(CHA, NW, KC).transpose(1, 0, 2)
    src_b = edge_index[0][EA:].reshape(CHB, NW, KC).transpose(1, 0, 2)
    dst_b = edge_index[1][EA:].reshape(CHB, NW, KC).transpose(1, 0, 2)
    ones128 = jnp.ones((KC, DD), jnp.float32)
    z128 = jnp.zeros((RPT, DD), jnp.float32)

    deg_p = _sc_degree(dst_r, ones128, z128)
    g1 = _tc_g1(x, W1, deg_p)
    S1 = _sc_aggregate(g1, src_r, dst_r, z128)
    g2 = _tc_mid(S1, g1, deg_p, W2, b1.reshape(1, -1))
    S2 = _sc_aggregate(g2, src_r, dst_r, z128)
    f, logits = _tc_final(S2, g2, deg_p, b2.reshape(1, -1), Wc,
                          bc.reshape(1, -1))
    Wa = Wp1[:DD].astype(jnp.bfloat16)
    Wb = Wp1[DD:].astype(jnp.bfloat16)
    W2b = Wp2.astype(jnp.bfloat16)
    fs_a, fd_a = _sc_pair_gather(f, src_a, dst_a, CHA)
    fs_b, fd_b = _sc_pair_gather(f, src_b, dst_b, CHB)
    e_a = _tc_edge_mlp_a(fs_a, fd_a, Wa, Wb, bp1.reshape(1, -1),
                         W2b, bp2.reshape(1, -1))
    edge_feats = _tc_edge_mlp_b(e_a, fs_b, fd_b, Wa, Wb, bp1.reshape(1, -1),
                                W2b, bp2.reshape(1, -1))
    return (f, edge_feats, logits, edge_index)


# speedup vs baseline: 1.2658x; 1.0006x over previous
import functools

import jax
import jax.numpy as jnp
from jax import lax
from jax.experimental import pallas as pl
from jax.experimental.pallas import tpu as pltpu
from jax.experimental.pallas import tpu_sc as plsc

NN = 10000
EE = 320000
DD = 128
NW = 32
EPT = EE // NW
KC = 80
CH = EPT // KC
NP = 10240
RPT = NP // 16

CHA = 63
CHB = CH - CHA
EA = NW * CHA * KC
EB = EE - EA
BKA = 2016
BKB = 1280

_mesh = plsc.VectorSubcoreMesh(core_axis_name="c", subcore_axis_name="s")


def _sc_degree(dst_r, ones128, z128):
    @functools.partial(
        pl.kernel, mesh=_mesh,
        out_type=jax.ShapeDtypeStruct((2, NP, DD), jnp.float32),
        scratch_types=[
            pltpu.VMEM((CH, KC), jnp.int32),
            pltpu.VMEM((KC, DD), jnp.float32),
            pltpu.VMEM_SHARED((NP, DD), jnp.float32),
            pltpu.SemaphoreType.DMA,
        ],
    )
    def k(dst_hbm, ones_hbm, z_hbm, out_hbm, didx_v, ones_v, acc_sh, sem):
        cid = lax.axis_index("c")
        sid = lax.axis_index("s")
        wid = sid * 2 + cid
        pltpu.sync_copy(z_hbm, acc_sh.at[pl.ds(sid * RPT, RPT)])
        pltpu.sync_copy(dst_hbm.at[wid], didx_v)
        pltpu.sync_copy(ones_hbm, ones_v)
        plsc.subcore_barrier()

        def body(j, c):
            pltpu.sync_copy(ones_v, acc_sh.at[didx_v.at[j]], add=True)
            return c

        lax.fori_loop(0, CH, body, 0)
        plsc.subcore_barrier()
        pltpu.sync_copy(acc_sh.at[pl.ds(sid * RPT, RPT)],
                        out_hbm.at[cid, pl.ds(sid * RPT, RPT)])

    return k(dst_r, ones128, z128)


def _sc_aggregate(g, src_r, dst_r, z128):
    @functools.partial(
        pl.kernel, mesh=_mesh,
        out_type=jax.ShapeDtypeStruct((2, NP, DD), jnp.float32),
        scratch_types=[
            pltpu.VMEM((CH, KC), jnp.int32),
            pltpu.VMEM((CH, KC), jnp.int32),
            pltpu.VMEM((KC, DD), jnp.float32),
            pltpu.VMEM_SHARED((NP, DD), jnp.float32),
            pltpu.SemaphoreType.DMA,
        ],
    )
    def k(g_hbm, src_hbm, dst_hbm, z_hbm, out_hbm,
          sidx_v, didx_v, rows_v, acc_sh, sem):
        cid = lax.axis_index("c")
        sid = lax.axis_index("s")
        wid = sid * 2 + cid
        pltpu.sync_copy(z_hbm, acc_sh.at[pl.ds(sid * RPT, RPT)])
        pltpu.sync_copy(src_hbm.at[wid], sidx_v)
        pltpu.sync_copy(dst_hbm.at[wid], didx_v)
        plsc.subcore_barrier()

        def body(j, c):
            pltpu.sync_copy(g_hbm.at[sidx_v.at[j]], rows_v)
            pltpu.sync_copy(rows_v, acc_sh.at[didx_v.at[j]], add=True)
            return c

        lax.fori_loop(0, CH, body, 0)
        plsc.subcore_barrier()
        pltpu.sync_copy(acc_sh.at[pl.ds(sid * RPT, RPT)],
                        out_hbm.at[cid, pl.ds(sid * RPT, RPT)])

    return k(g, src_r, dst_r, z128)


def _sc_pair_gather(f, src_r, dst_r, ch):
    ne = NW * ch * KC

    @functools.partial(
        pl.kernel, mesh=_mesh,
        out_type=(jax.ShapeDtypeStruct((ne, DD), jnp.float32),
                  jax.ShapeDtypeStruct((ne, DD), jnp.float32)),
        scratch_types=[
            pltpu.VMEM((ch, KC), jnp.int32),
            pltpu.VMEM((ch, KC), jnp.int32),
            pltpu.VMEM((KC, DD), jnp.float32),
            pltpu.VMEM((KC, DD), jnp.float32),
            pltpu.VMEM((KC, DD), jnp.float32),
            pltpu.VMEM((KC, DD), jnp.float32),
            pltpu.SemaphoreType.DMA,
            pltpu.SemaphoreType.DMA,
        ],
    )
    def k(f_hbm, src_hbm, dst_hbm, os_hbm, od_hbm,
          sidx_v, didx_v, rs0_v, rd0_v, rs1_v, rd1_v, sem0, sem1):
        cid = lax.axis_index("c")
        sid = lax.axis_index("s")
        wid = sid * 2 + cid
        base0 = wid * KC
        pltpu.sync_copy(src_hbm.at[wid], sidx_v)
        pltpu.sync_copy(dst_hbm.at[wid], didx_v)

        def body(jj, c):
            j = 2 * jj
            cs0 = pltpu.async_copy(f_hbm.at[sidx_v.at[j]], rs0_v, sem0)
            cd0 = pltpu.async_copy(f_hbm.at[didx_v.at[j]], rd0_v, sem0)
            cs1 = pltpu.async_copy(f_hbm.at[sidx_v.at[j + 1]], rs1_v, sem1)
            cd1 = pltpu.async_copy(f_hbm.at[didx_v.at[j + 1]], rd1_v, sem1)
            base = pl.multiple_of(base0 + j * NW * KC, 8)
            base1 = pl.multiple_of(base0 + (j + 1) * NW * KC, 8)
            cs0.wait()
            cd0.wait()
            pltpu.sync_copy(rs0_v, os_hbm.at[pl.ds(base, KC)])
            pltpu.sync_copy(rd0_v, od_hbm.at[pl.ds(base, KC)])
            cs1.wait()
            cd1.wait()
            pltpu.sync_copy(rs1_v, os_hbm.at[pl.ds(base1, KC)])
            pltpu.sync_copy(rd1_v, od_hbm.at[pl.ds(base1, KC)])
            return c

        lax.fori_loop(0, ch // 2, body, 0)
        if ch % 2:
            base = pl.multiple_of(base0 + (ch - 1) * NW * KC, 8)
            cs = pltpu.async_copy(f_hbm.at[sidx_v.at[ch - 1]], rs0_v, sem0)
            cd = pltpu.async_copy(f_hbm.at[didx_v.at[ch - 1]], rd0_v, sem0)
            cs.wait()
            cd.wait()
            pltpu.sync_copy(rs0_v, os_hbm.at[pl.ds(base, KC)])
            pltpu.sync_copy(rd0_v, od_hbm.at[pl.ds(base, KC)])

    return k(f, src_r, dst_r)


def _dinv_col(dp_ref):
    deg = dp_ref[0, :, 0:1] + dp_ref[1, :, 0:1] + 1.0
    return lax.rsqrt(deg)


def _tc_h1(x, W1):
    R = 1000

    def body(x_ref, w_ref, o_ref):
        o_ref[...] = jnp.dot(x_ref[...], w_ref[...],
                             preferred_element_type=jnp.float32)

    return pl.pallas_call(
        body,
        grid=(NN // R,),
        in_specs=[
            pl.BlockSpec((R, DD), lambda i: (i, 0)),
            pl.BlockSpec((DD, DD), lambda i: (0, 0)),
        ],
        out_specs=pl.BlockSpec((R, DD), lambda i: (i, 0)),
        out_shape=jax.ShapeDtypeStruct((NN, DD), jnp.float32),
    )(x, W1)


def _tc_scale(h1, deg_p):
    R = 2000

    def body(h_ref, dp_ref, o_ref):
        o_ref[...] = h_ref[...] * _dinv_col(dp_ref)

    return pl.pallas_call(
        body,
        grid=(NN // R,),
        in_specs=[
            pl.BlockSpec((R, DD), lambda i: (i, 0)),
            pl.BlockSpec((2, R, DD), lambda i: (0, i, 0)),
        ],
        out_specs=pl.BlockSpec((R, DD), lambda i: (i, 0)),
        out_shape=jax.ShapeDtypeStruct((NN, DD), jnp.float32),
    )(h1, deg_p)


def _tc_mid(S_p, g1, deg_p, W2, b1):
    R = 1000

    def body(sp_ref, g_ref, dp_ref, w_ref, b_ref, o_ref):
        col = _dinv_col(dp_ref)
        x1 = (sp_ref[0] + sp_ref[1] + g_ref[...]) * col + b_ref[...]
        x1 = jnp.maximum(x1, 0.0)
        h = jnp.dot(x1, w_ref[...], preferred_element_type=jnp.float32)
        o_ref[...] = h * col

    return pl.pallas_call(
        body,
        grid=(NN // R,),
        in_specs=[
            pl.BlockSpec((2, R, DD), lambda i: (0, i, 0)),
            pl.BlockSpec((R, DD), lambda i: (i, 0)),
            pl.BlockSpec((2, R, DD), lambda i: (0, i, 0)),
            pl.BlockSpec((DD, DD), lambda i: (0, 0)),
            pl.BlockSpec((1, DD), lambda i: (0, 0)),
        ],
        out_specs=pl.BlockSpec((R, DD), lambda i: (i, 0)),
        out_shape=jax.ShapeDtypeStruct((NN, DD), jnp.float32),
    )(S_p, g1, deg_p, W2, b1)


def _tc_final(S_p, g2, deg_p, b2, Wc, bc):
    R = 1000
    ncls = Wc.shape[1]

    def body(sp_ref, g_ref, dp_ref, b_ref, wc_ref, bc_ref, f_ref, lg_ref):
        col = _dinv_col(dp_ref)
        f = (sp_ref[0] + sp_ref[1] + g_ref[...]) * col + b_ref[...]
        f_ref[...] = f
        lg_ref[...] = jnp.dot(f, wc_ref[...],
                              preferred_element_type=jnp.float32) + bc_ref[...]

    return pl.pallas_call(
        body,
        grid=(NN // R,),
        in_specs=[
            pl.BlockSpec((2, R, DD), lambda i: (0, i, 0)),
            pl.BlockSpec((R, DD), lambda i: (i, 0)),
            pl.BlockSpec((2, R, DD), lambda i: (0, i, 0)),
            pl.BlockSpec((1, DD), lambda i: (0, 0)),
            pl.BlockSpec((DD, ncls), lambda i: (0, 0)),
            pl.BlockSpec((1, ncls), lambda i: (0, 0)),
        ],
        out_specs=[
            pl.BlockSpec((R, DD), lambda i: (i, 0)),
            pl.BlockSpec((R, ncls), lambda i: (i, 0)),
        ],
        out_shape=[
            jax.ShapeDtypeStruct((NN, DD), jnp.float32),
            jax.ShapeDtypeStruct((NN, ncls), jnp.float32),
        ],
    )(S_p, g2, deg_p, b2, Wc, bc)


def _edge_mlp_body(fs_ref, fd_ref, a_ref, b_ref, b1_ref, w2_ref, b2_ref, o_ref):
    h = jnp.dot(fs_ref[...].astype(jnp.bfloat16), a_ref[...],
                preferred_element_type=jnp.float32)
    h = h + jnp.dot(fd_ref[...].astype(jnp.bfloat16), b_ref[...],
                    preferred_element_type=jnp.float32)
    h = jnp.maximum(h + b1_ref[...], 0.0).astype(jnp.bfloat16)
    o_ref[...] = jnp.dot(h, w2_ref[...],
                         preferred_element_type=jnp.float32) + b2_ref[...]


def _tc_edge_mlp_a(fs, fd, Wa, Wb, bp1, Wp2, bp2):
    DH = Wa.shape[1]

    return pl.pallas_call(
        _edge_mlp_body,
        grid=(EA // BKA,),
        in_specs=[
            pl.BlockSpec((BKA, DD), lambda i: (i, 0)),
            pl.BlockSpec((BKA, DD), lambda i: (i, 0)),
            pl.BlockSpec((DD, DH), lambda i: (0, 0)),
            pl.BlockSpec((DD, DH), lambda i: (0, 0)),
            pl.BlockSpec((1, DH), lambda i: (0, 0)),
            pl.BlockSpec((DH, DD), lambda i: (0, 0)),
            pl.BlockSpec((1, DD), lambda i: (0, 0)),
        ],
        out_specs=pl.BlockSpec((BKA, DD), lambda i: (i, 0)),
        out_shape=jax.ShapeDtypeStruct((EE, DD), jnp.float32),
    )(fs, fd, Wa, Wb, bp1, Wp2, bp2)


def _tc_edge_mlp_b(prev, fs, fd, Wa, Wb, bp1, Wp2, bp2):
    DH = Wa.shape[1]
    OFF = EA // BKB

    def body(p_ref, fs_ref, fd_ref, a_ref, b_ref, b1_ref, w2_ref, b2_ref,
             o_ref):
        _edge_mlp_body(fs_ref, fd_ref, a_ref, b_ref, b1_ref, w2_ref, b2_ref,
                       o_ref)

    return pl.pallas_call(
        body,
        grid=(EB // BKB,),
        in_specs=[
            pl.BlockSpec(memory_space=pl.ANY),
            pl.BlockSpec((BKB, DD), lambda i: (i, 0)),
            pl.BlockSpec((BKB, DD), lambda i: (i, 0)),
            pl.BlockSpec((DD, DH), lambda i: (0, 0)),
            pl.BlockSpec((DD, DH), lambda i: (0, 0)),
            pl.BlockSpec((1, DH), lambda i: (0, 0)),
            pl.BlockSpec((DH, DD), lambda i: (0, 0)),
            pl.BlockSpec((1, DD), lambda i: (0, 0)),
        ],
        out_specs=pl.BlockSpec((BKB, DD), lambda i: (i + OFF, 0)),
        out_shape=jax.ShapeDtypeStruct((EE, DD), jnp.float32),
        input_output_aliases={0: 0},
    )(prev, fs, fd, Wa, Wb, bp1, Wp2, bp2)


def kernel(x, edge_index, W1, b1, W2, b2, Wp1, bp1, Wp2, bp2, Wc, bc):
    src_r = edge_index[0].reshape(NW, CH, KC)
    dst_r = edge_index[1].reshape(NW, CH, KC)
    src_a = edge_index[0][:EA].reshape(CHA, NW, KC).transpose(1, 0, 2)
    dst_a = edge_index[1][:EA].reshape(CHA, NW, KC).transpose(1, 0, 2)
    src_b = edge_index[0][EA:].reshape(CHB, NW, KC).transpose(1, 0, 2)
    dst_b = edge_index[1][EA:].reshape(CHB, NW, KC).transpose(1, 0, 2)
    ones128 = jnp.ones((KC, DD), jnp.float32)
    z128 = jnp.zeros((RPT, DD), jnp.float32)

    deg_p = _sc_degree(dst_r, ones128, z128)
    g1 = _tc_scale(_tc_h1(x, W1), deg_p)
    S1 = _sc_aggregate(g1, src_r, dst_r, z128)
    g2 = _tc_mid(S1, g1, deg_p, W2, b1.reshape(1, -1))
    S2 = _sc_aggregate(g2, src_r, dst_r, z128)
    f, logits = _tc_final(S2, g2, deg_p, b2.reshape(1, -1), Wc,
                          bc.reshape(1, -1))
    Wa = Wp1[:DD].astype(jnp.bfloat16)
    Wb = Wp1[DD:].astype(jnp.bfloat16)
    W2b = Wp2.astype(jnp.bfloat16)
    fs_a, fd_a = _sc_pair_gather(f, src_a, dst_a, CHA)
    fs_b, fd_b = _sc_pair_gather(f, src_b, dst_b, CHB)
    e_a = _tc_edge_mlp_a(fs_a, fd_a, Wa, Wb, bp1.reshape(1, -1),
                         W2b, bp2.reshape(1, -1))
    edge_feats = _tc_edge_mlp_b(e_a, fs_b, fd_b, Wa, Wb, bp1.reshape(1, -1),
                                W2b, bp2.reshape(1, -1))
    return (f, edge_feats, logits, edge_index)
